# baseline (speedup 1.0000x reference)
import functools

import jax
import jax.numpy as jnp
from jax import lax
from jax.experimental import pallas as pl
from jax.experimental.pallas import tpu as pltpu
from jax.experimental.pallas import tpu_sc as plsc

B = 16384
L = 200
D = 64
H = 128
C = 20

NUM_CORES = 2
NUM_SUBCORES = 16
NW = NUM_CORES * NUM_SUBCORES
BAGS_PER_W = B // NW
NG = BAGS_PER_W
NVREG = D // 16

NROWS = 4
NIDX = 8

_CHUNKS = [(0, 128), (128, 72)]


def _bag_body(tokens_hbm, table_hbm, out_hbm, idx_v, rows_v, out_v, sem_g, sem_t):
    wid = lax.axis_index("s") * NUM_CORES + lax.axis_index("c")
    tok_base = wid * BAGS_PER_W * L

    def tok_slice(g):
        return tokens_hbm.at[pl.ds(tok_base + g * L, L)]

    def fire_gathers(gslot, islot):
        for off, sz in _CHUNKS:
            pltpu.async_copy(
                table_hbm.at[idx_v.at[islot].at[pl.ds(off, sz)]],
                rows_v.at[gslot].at[pl.ds(off, sz)],
                sem_g,
            )

    def drain_gathers(gslot):
        for off, sz in _CHUNKS:
            pltpu.make_async_copy(
                table_hbm.at[pl.ds(0, sz)],
                rows_v.at[gslot].at[pl.ds(off, sz)],
                sem_g,
            ).wait()

    def drain_tokens(islot):
        pltpu.make_async_copy(
            tokens_hbm.at[pl.ds(0, L)],
            idx_v.at[islot],
            sem_t,
        ).wait()

    for h in range(3):
        pltpu.sync_copy(tok_slice(h), idx_v.at[h])
    fire_gathers(0, 0)
    fire_gathers(1, 1)
    pltpu.async_copy(tok_slice(3), idx_v.at[3], sem_t)
    pltpu.async_copy(tok_slice(4), idx_v.at[4], sem_t)

    def outer(i, carry):
        for j in range(8):
            g = i * 8 + j
            gslot = j % NROWS
            gslot_n2 = (j + 2) % NROWS
            islot_n2, islot_n5 = (j + 2) % NIDX, (j + 5) % NIDX

            @pl.when(jnp.logical_and(g >= 1, g < NG - 2))
            def _():
                drain_tokens(islot_n2)

            @pl.when(g < NG - 2)
            def _():
                fire_gathers(gslot_n2, islot_n2)

            @pl.when(g < NG - 5)
            def _():
                pltpu.async_copy(tok_slice(g + 5), idx_v.at[islot_n5], sem_t)

            drain_gathers(gslot)

            def red_body(r, acc, _gslot=gslot):
                return tuple(
                    acc[c] + rows_v[_gslot, r, pl.ds(c * 16, 16)]
                    for c in range(NVREG)
                )
            acc = lax.fori_loop(
                0, L, red_body,
                tuple(jnp.zeros((16,), jnp.float32) for _ in range(NVREG)),
                unroll=8,
            )
            for c in range(NVREG):
                out_v[g, pl.ds(c * 16, 16)] = acc[c] * (1.0 / L)
        return carry

    lax.fori_loop(0, NG // 8, outer, 0)
    pltpu.sync_copy(out_v, out_hbm.at[pl.ds(wid * BAGS_PER_W, BAGS_PER_W)])


_bag_call = functools.partial(
    pl.kernel,
    out_type=jax.ShapeDtypeStruct((B, D), jnp.float32),
    mesh=plsc.VectorSubcoreMesh(core_axis_name="c", subcore_axis_name="s"),
    scratch_types=[
        pltpu.VMEM((NIDX, L), jnp.int32),
        pltpu.VMEM((NROWS, L, D), jnp.float32),
        pltpu.VMEM((BAGS_PER_W, D), jnp.float32),
        pltpu.SemaphoreType.DMA,
        pltpu.SemaphoreType.DMA,
    ],
    compiler_params=pltpu.CompilerParams(use_tc_tiling_on_sc=False),
)(_bag_body)


def _mlp_body(x_ref, w1_ref, b1_ref, w2_ref, b2_ref, o_ref):
    x = x_ref[...]
    h = jnp.dot(x, w1_ref[...], preferred_element_type=jnp.float32)
    h = jnp.maximum(h + b1_ref[...], 0.0)
    o_ref[...] = jnp.dot(h, w2_ref[...], preferred_element_type=jnp.float32) + b2_ref[...]


def _mlp_call(x, w1, b1, w2, b2):
    bt = 1024
    grid = (B // bt,)
    return pl.pallas_call(
        _mlp_body,
        grid=grid,
        in_specs=[
            pl.BlockSpec((bt, D), lambda i: (i, 0)),
            pl.BlockSpec((D, H), lambda i: (0, 0)),
            pl.BlockSpec((1, H), lambda i: (0, 0)),
            pl.BlockSpec((H, C), lambda i: (0, 0)),
            pl.BlockSpec((1, C), lambda i: (0, 0)),
        ],
        out_specs=pl.BlockSpec((bt, C), lambda i: (i, 0)),
        out_shape=jax.ShapeDtypeStruct((B, C), jnp.float32),
    )(x, w1, b1, w2, b2)


def kernel(tokens, emb_table, W1, b1, W2, b2):
    bags = _bag_call(tokens.reshape(-1), emb_table)
    return _mlp_call(bags, W1, b1.reshape(1, H), W2, b2.reshape(1, C))


# speedup vs baseline: 1.0385x; 1.0385x over previous
import functools

import jax
import jax.numpy as jnp
from jax import lax
from jax.experimental import pallas as pl
from jax.experimental.pallas import tpu as pltpu
from jax.experimental.pallas import tpu_sc as plsc

B = 16384
L = 200
D = 64
H = 128
C = 20

NUM_CORES = 2
NUM_SUBCORES = 16
NW = NUM_CORES * NUM_SUBCORES
BAGS_PER_W = B // NW
NG = BAGS_PER_W
NVREG = D // 16

NROWS = 4
NIDX = 8

_CHUNKS = [(0, 128), (128, 72)]


def _bag_body(tokens_hbm, table_hbm, out_hbm, idx_v, rows_v, out_v, sem_g, sem_t):
    wid = lax.axis_index("s") * NUM_CORES + lax.axis_index("c")
    tok_base = wid * BAGS_PER_W * L

    def tok_slice(g):
        return tokens_hbm.at[pl.ds(tok_base + g * L, L)]

    def fire_gathers(gslot, islot):
        for off, sz in _CHUNKS:
            pltpu.async_copy(
                table_hbm.at[idx_v.at[islot].at[pl.ds(off, sz)]],
                rows_v.at[gslot].at[pl.ds(off, sz)],
                sem_g,
            )

    def drain_gathers(gslot):
        for off, sz in _CHUNKS:
            pltpu.make_async_copy(
                table_hbm.at[pl.ds(0, sz)],
                rows_v.at[gslot].at[pl.ds(off, sz)],
                sem_g,
            ).wait()

    def drain_tokens(islot):
        pltpu.make_async_copy(
            tokens_hbm.at[pl.ds(0, L)],
            idx_v.at[islot],
            sem_t,
        ).wait()

    for h in range(3):
        pltpu.sync_copy(tok_slice(h), idx_v.at[h])
    for h in range(3):
        fire_gathers(h, h)
    for h in range(3, 6):
        pltpu.async_copy(tok_slice(h), idx_v.at[h], sem_t)

    def outer(i, carry):
        for j in range(8):
            g = i * 8 + j
            gslot = j % NROWS
            gslot_n3 = (j + 3) % NROWS
            islot_n3, islot_n6 = (j + 3) % NIDX, (j + 6) % NIDX

            @pl.when(g < NG - 3)
            def _():
                drain_tokens(islot_n3)
                fire_gathers(gslot_n3, islot_n3)

            @pl.when(g < NG - 6)
            def _():
                pltpu.async_copy(tok_slice(g + 6), idx_v.at[islot_n6], sem_t)

            drain_gathers(gslot)

            def red_body(r, acc, _gslot=gslot):
                return tuple(
                    acc[c] + rows_v[_gslot, r, pl.ds(c * 16, 16)]
                    for c in range(NVREG)
                )
            acc = lax.fori_loop(
                0, L, red_body,
                tuple(jnp.zeros((16,), jnp.float32) for _ in range(NVREG)),
                unroll=8,
            )
            for c in range(NVREG):
                out_v[g, pl.ds(c * 16, 16)] = acc[c] * (1.0 / L)
        return carry

    lax.fori_loop(0, NG // 8, outer, 0)
    pltpu.sync_copy(out_v, out_hbm.at[pl.ds(wid * BAGS_PER_W, BAGS_PER_W)])


_bag_call = functools.partial(
    pl.kernel,
    out_type=jax.ShapeDtypeStruct((B, D), jnp.float32),
    mesh=plsc.VectorSubcoreMesh(core_axis_name="c", subcore_axis_name="s"),
    scratch_types=[
        pltpu.VMEM((NIDX, L), jnp.int32),
        pltpu.VMEM((NROWS, L, D), jnp.float32),
        pltpu.VMEM((BAGS_PER_W, D), jnp.float32),
        pltpu.SemaphoreType.DMA,
        pltpu.SemaphoreType.DMA,
    ],
    compiler_params=pltpu.CompilerParams(use_tc_tiling_on_sc=False),
)(_bag_body)


def _mlp_body(x_ref, w1_ref, b1_ref, w2_ref, b2_ref, o_ref):
    x = x_ref[...]
    h = jnp.dot(x, w1_ref[...], preferred_element_type=jnp.float32)
    h = jnp.maximum(h + b1_ref[...], 0.0)
    o_ref[...] = jnp.dot(h, w2_ref[...], preferred_element_type=jnp.float32) + b2_ref[...]


def _mlp_call(x, w1, b1, w2, b2):
    bt = 1024
    grid = (B // bt,)
    return pl.pallas_call(
        _mlp_body,
        grid=grid,
        in_specs=[
            pl.BlockSpec((bt, D), lambda i: (i, 0)),
            pl.BlockSpec((D, H), lambda i: (0, 0)),
            pl.BlockSpec((1, H), lambda i: (0, 0)),
            pl.BlockSpec((H, C), lambda i: (0, 0)),
            pl.BlockSpec((1, C), lambda i: (0, 0)),
        ],
        out_specs=pl.BlockSpec((bt, C), lambda i: (i, 0)),
        out_shape=jax.ShapeDtypeStruct((B, C), jnp.float32),
    )(x, w1, b1, w2, b2)


def kernel(tokens, emb_table, W1, b1, W2, b2):
    bags = _bag_call(tokens.reshape(-1), emb_table)
    return _mlp_call(bags, W1, b1.reshape(1, H), W2, b2.reshape(1, C))
